# Initial kernel scaffold; baseline (speedup 1.0000x reference)
import jax
import jax.numpy as jnp
from jax.experimental import pallas as pl


def kernel(x, activation_idx, batch_idx, num_layers, W_phi1, b_phi1, W_phi2, b_phi2, W_rho1, b_rho1, W_rho2, b_rho2):
    raise NotImplementedError("write your pallas kernel here")


# speedup vs baseline: 2.6894x; 2.6894x over previous
import jax
import jax.numpy as jnp
from jax.experimental import pallas as pl
from jax.experimental.pallas import tpu as pltpu

_SLOTS = 16


def _body(x_ref, act_ref, bat_ref, nl_ref,
          w1_ref, b1_ref, w2_ref, b2_ref,
          wr1_ref, br1_ref, wr2_ref, br2_ref,
          out_ref, acc_ref, cnt_ref):
    i = pl.program_id(0)
    R = x_ref.shape[0]
    B = out_ref.shape[0]

    @pl.when(i == 0)
    def _init():
        acc_ref[...] = jnp.zeros_like(acc_ref)
        cnt_ref[...] = jnp.zeros_like(cnt_ref)

    a = jnp.maximum(
        jnp.dot(x_ref[...], w1_ref[...], preferred_element_type=jnp.float32)
        + b1_ref[...], 0.0)

    act = act_ref[0]
    bat = bat_ref[0]
    bmin = bat_ref[0, 0, 0]
    bmax = bat_ref[0, 0, R - 1]

    slot_iota = jax.lax.broadcasted_iota(jnp.int32, (_SLOTS, R), 0)

    def batch_body(b, carry):
        onehot = jnp.where((act == slot_iota) & (bat == b), 1.0, 0.0)
        partial = jnp.dot(onehot, a, preferred_element_type=jnp.float32)
        c = jnp.sum(onehot, axis=1, keepdims=True)
        off = b * _SLOTS
        acc_ref[pl.ds(off, _SLOTS), :] += partial
        cnt_ref[pl.ds(off, _SLOTS), :] += jnp.broadcast_to(c, (_SLOTS, 128))
        return carry

    jax.lax.fori_loop(bmin, bmax + 1, batch_body, 0)

    @pl.when(i == pl.num_programs(0) - 1)
    def _finish():
        nl = nl_ref[0, 0]
        xsum = (jnp.dot(acc_ref[...], w2_ref[...],
                        preferred_element_type=jnp.float32)
                + cnt_ref[...] * b2_ref[...])
        r = jnp.maximum(
            jnp.dot(xsum, wr1_ref[...], preferred_element_type=jnp.float32)
            + br1_ref[...], 0.0)
        r = (jnp.dot(r, wr2_ref[...], preferred_element_type=jnp.float32)
             + br2_ref[...])
        r3 = r.reshape(B, _SLOTS, r.shape[-1])
        lidx = jax.lax.broadcasted_iota(jnp.int32, r3.shape, 1)
        out_ref[...] = jnp.sum(jnp.where(lidx < nl, r3, 0.0), axis=1)


def kernel(x, activation_idx, batch_idx, num_layers,
           W_phi1, b_phi1, W_phi2, b_phi2,
           W_rho1, b_rho1, W_rho2, b_rho2):
    N, D_IN = x.shape
    D_OUT = W_rho2.shape[1]
    B = 64

    R = 512
    while N % R:
        R //= 2
    G = N // R

    act3 = activation_idx.astype(jnp.int32).reshape(G, 1, R)
    bat3 = batch_idx.astype(jnp.int32).reshape(G, 1, R)
    nl = jnp.asarray(num_layers, jnp.int32).reshape(1, 1)

    full = lambda shp: pl.BlockSpec(shp, lambda i: (0,) * len(shp))
    out = pl.pallas_call(
        _body,
        grid=(G,),
        in_specs=[
            pl.BlockSpec((R, D_IN), lambda i: (i, 0)),
            pl.BlockSpec((1, 1, R), lambda i: (i, 0, 0)),
            pl.BlockSpec((1, 1, R), lambda i: (i, 0, 0)),
            pl.BlockSpec(memory_space=pltpu.SMEM),
            full((D_IN, D_IN)), full((1, D_IN)),
            full((D_IN, D_IN)), full((1, D_IN)),
            full((D_IN, D_IN)), full((1, D_IN)),
            full((D_IN, D_OUT)), full((1, D_OUT)),
        ],
        out_specs=pl.BlockSpec((B, D_OUT), lambda i: (0, 0)),
        out_shape=jax.ShapeDtypeStruct((B, D_OUT), jnp.float32),
        scratch_shapes=[
            pltpu.VMEM((B * _SLOTS, D_IN), jnp.float32),
            pltpu.VMEM((B * _SLOTS, D_IN), jnp.float32),
        ],
        compiler_params=pltpu.CompilerParams(
            dimension_semantics=("arbitrary",)),
    )(x, act3, bat3, nl,
      W_phi1, b_phi1.reshape(1, -1), W_phi2, b_phi2.reshape(1, -1),
      W_rho1, b_rho1.reshape(1, -1), W_rho2, b_rho2.reshape(1, -1))
    return out
